# baseline (speedup 1.0000x reference)
import functools

import jax
import jax.numpy as jnp
from jax import lax
from jax.experimental import pallas as pl
from jax.experimental.pallas import tpu as pltpu
from jax.experimental.pallas import tpu_sc as plsc

D = 64
SCALE = float(D) ** 0.5

B = 4096 * 200
NC = 2
NS = 16
NW = NC * NS
PER_W = B // NW
CHUNK = 128
N_CHUNKS = PER_W // CHUNK

_mesh = plsc.VectorSubcoreMesh(core_axis_name="c", subcore_axis_name="s")


@functools.partial(
    pl.kernel,
    mesh=_mesh,
    out_type=jax.ShapeDtypeStruct((B, D), jnp.float32),
    scratch_types=[
        pltpu.VMEM((CHUNK,), jnp.int32),
        pltpu.VMEM((CHUNK, D), jnp.float32),
        pltpu.SemaphoreType.DMA,
    ],
    compiler_params=pltpu.CompilerParams(use_tc_tiling_on_sc=False),
)
def _embed(idx_hbm, table_hbm, out_hbm, idx_v, rows_v, sem):
    wid = lax.axis_index("s") * NC + lax.axis_index("c")
    base = wid * PER_W

    def chunk_body(i, carry):
        start = base + i * CHUNK
        pltpu.sync_copy(idx_hbm.at[pl.ds(start, CHUNK)], idx_v)
        pltpu.async_copy(table_hbm.at[idx_v], rows_v, sem).wait()

        def grp_body(g, c2):
            idx16 = idx_v[pl.ds(g * 16, 16)]
            s = jnp.where(idx16 != 0, SCALE, 0.0).astype(jnp.float32)
            for j in range(16):
                sj = s[j]
                r = g * 16 + j
                for c in range(D // 16):
                    sl = pl.ds(c * 16, 16)
                    rows_v[r, sl] = rows_v[r, sl] * sj
            return c2

        lax.fori_loop(0, CHUNK // 16, grp_body, 0)
        pltpu.sync_copy(rows_v, out_hbm.at[pl.ds(start, CHUNK)])
        return carry

    lax.fori_loop(0, N_CHUNKS, chunk_body, 0)


def kernel(input, lookup_table):
    idx = input.reshape(-1).astype(jnp.int32)
    out = _embed(idx, lookup_table)
    return out.reshape(input.shape + (D,))


# speedup vs baseline: 1.5072x; 1.5072x over previous
import functools

import jax
import jax.numpy as jnp
from jax import lax
from jax.experimental import pallas as pl
from jax.experimental.pallas import tpu as pltpu
from jax.experimental.pallas import tpu_sc as plsc

D = 64
SCALE = float(D) ** 0.5

B = 4096 * 200
NC = 2
NS = 16
NW = NC * NS
PER_W = B // NW
CHUNK = 128
N_CHUNKS = PER_W // CHUNK
NBUF = 4

_mesh = plsc.VectorSubcoreMesh(core_axis_name="c", subcore_axis_name="s")


@functools.partial(
    pl.kernel,
    mesh=_mesh,
    out_type=jax.ShapeDtypeStruct((B, D), jnp.float32),
    scratch_types=[
        pltpu.VMEM((N_CHUNKS, CHUNK), jnp.int32),
        pltpu.VMEM((NBUF, CHUNK, D), jnp.float32),
        pltpu.SemaphoreType.DMA((NBUF,)),
        pltpu.SemaphoreType.DMA((NBUF,)),
    ],
    compiler_params=pltpu.CompilerParams(use_tc_tiling_on_sc=False),
)
def _embed(idx_hbm, table_hbm, out_hbm, idx_v, rows_v, gsem, osem):
    wid = lax.axis_index("s") * NC + lax.axis_index("c")
    out_base = wid * PER_W

    pltpu.sync_copy(idx_hbm.at[wid], idx_v)

    def fire_gather(chunk, b):
        pltpu.async_copy(table_hbm.at[idx_v.at[chunk]], rows_v.at[b], gsem.at[b])

    def wait_gather(chunk, b):
        pltpu.make_async_copy(
            table_hbm.at[idx_v.at[chunk]], rows_v.at[b], gsem.at[b]
        ).wait()

    def fire_scatter(chunk, b):
        dst = out_hbm.at[pl.ds(out_base + chunk * CHUNK, CHUNK)]
        pltpu.async_copy(rows_v.at[b], dst, osem.at[b])

    def wait_scatter(chunk, b):
        dst = out_hbm.at[pl.ds(out_base + chunk * CHUNK, CHUNK)]
        pltpu.make_async_copy(rows_v.at[b], dst, osem.at[b]).wait()

    def compute(chunk, b):
        def grp_body(g, c2):
            idx16 = idx_v[chunk, pl.ds(g * 16, 16)]
            s = jnp.where(idx16 != 0, SCALE, 0.0).astype(jnp.float32)
            for j in range(16):
                sj = s[j]
                r = g * 16 + j
                for c in range(D // 16):
                    sl = pl.ds(c * 16, 16)
                    rows_v[b, r, sl] = rows_v[b, r, sl] * sj
            return c2

        lax.fori_loop(0, CHUNK // 16, grp_body, 0, unroll=2)

    for b in range(NBUF - 1):
        fire_gather(b, b)

    def step(k, carry):
        for u in range(NBUF):
            i = k * NBUF + u
            b = u
            wait_gather(i, b)
            compute(i, b)
            fire_scatter(i, b)
            nxt = i + (NBUF - 1)
            bn = (u + NBUF - 1) % NBUF

            @pl.when(nxt < N_CHUNKS)
            def _():
                @pl.when(i >= 1)
                def _():
                    wait_scatter(i - 1, bn)

                fire_gather(nxt, bn)

        return carry

    lax.fori_loop(0, N_CHUNKS // NBUF, step, 0)

    for u in range(NBUF):
        chunk = N_CHUNKS - NBUF + u
        wait_scatter(chunk, chunk % NBUF)


def kernel(input, lookup_table):
    idx = input.reshape(NW, N_CHUNKS, CHUNK).astype(jnp.int32)
    out = _embed(idx, lookup_table)
    return out.reshape(input.shape + (D,))


# speedup vs baseline: 1.5092x; 1.0013x over previous
import functools

import jax
import jax.numpy as jnp
from jax import lax
from jax.experimental import pallas as pl
from jax.experimental.pallas import tpu as pltpu
from jax.experimental.pallas import tpu_sc as plsc

D = 64
SCALE = float(D) ** 0.5

R = 4096
T = 200
NC = 2
NS = 16
NW = NC * NS
ROWS_W = R // NW
G0 = 128
G1 = T - G0
NBUF = 4

_mesh = plsc.VectorSubcoreMesh(core_axis_name="c", subcore_axis_name="s")


@functools.partial(
    pl.kernel,
    mesh=_mesh,
    out_type=jax.ShapeDtypeStruct((R, T, D), jnp.float32),
    scratch_types=[
        pltpu.VMEM((ROWS_W, T), jnp.int32),
        pltpu.VMEM((NBUF, T, D), jnp.float32),
        pltpu.SemaphoreType.DMA((NBUF,)),
        pltpu.SemaphoreType.DMA((NBUF,)),
    ],
    compiler_params=pltpu.CompilerParams(use_tc_tiling_on_sc=False),
)
def _embed(idx_hbm, table_hbm, out_hbm, idx_v, rows_v, gsem, osem):
    wid = lax.axis_index("s") * NC + lax.axis_index("c")
    row0 = wid * ROWS_W

    pltpu.sync_copy(idx_hbm.at[pl.ds(row0, ROWS_W)], idx_v)

    def fire_gather(r, b):
        pltpu.async_copy(
            table_hbm.at[idx_v.at[r, pl.ds(0, G0)]],
            rows_v.at[b, pl.ds(0, G0)],
            gsem.at[b],
        )
        pltpu.async_copy(
            table_hbm.at[idx_v.at[r, pl.ds(G0, G1)]],
            rows_v.at[b, pl.ds(G0, G1)],
            gsem.at[b],
        )

    def wait_gather(r, b):
        pltpu.make_async_copy(
            table_hbm.at[idx_v.at[r, pl.ds(0, G0)]],
            rows_v.at[b, pl.ds(0, G0)],
            gsem.at[b],
        ).wait()
        pltpu.make_async_copy(
            table_hbm.at[idx_v.at[r, pl.ds(G0, G1)]],
            rows_v.at[b, pl.ds(G0, G1)],
            gsem.at[b],
        ).wait()

    def fire_scatter(r, b):
        pltpu.async_copy(rows_v.at[b], out_hbm.at[row0 + r], osem.at[b])

    def wait_scatter(r, b):
        pltpu.make_async_copy(rows_v.at[b], out_hbm.at[row0 + r], osem.at[b]).wait()

    def scale16(b, s, tok0, n):
        for j in range(n):
            sj = s[16 - n + j]
            t = tok0 + j
            for c in range(D // 16):
                sl = pl.ds(c * 16, 16)
                rows_v[b, t, sl] = rows_v[b, t, sl] * sj

    def compute(r, b):
        def grp_body(g, c2):
            idx16 = idx_v[r, pl.ds(g * 16, 16)]
            s = jnp.where(idx16 != 0, SCALE, 0.0).astype(jnp.float32)
            for j in range(16):
                sj = s[j]
                t = g * 16 + j
                for c in range(D // 16):
                    sl = pl.ds(c * 16, 16)
                    rows_v[b, t, sl] = rows_v[b, t, sl] * sj
            return c2

        lax.fori_loop(0, (T // 16), grp_body, 0, unroll=2)
        idx16 = idx_v[r, pl.ds(T - 16, 16)]
        s = jnp.where(idx16 != 0, SCALE, 0.0).astype(jnp.float32)
        scale16(b, s, (T // 16) * 16, T - (T // 16) * 16)

    for b in range(NBUF - 1):
        fire_gather(b, b)

    def step(k, carry):
        for u in range(NBUF):
            i = k * NBUF + u
            b = u
            wait_gather(i, b)
            compute(i, b)
            fire_scatter(i, b)
            nxt = i + (NBUF - 1)
            bn = (u + NBUF - 1) % NBUF

            @pl.when(nxt < ROWS_W)
            def _():
                @pl.when(i >= 1)
                def _():
                    wait_scatter(i - 1, bn)

                fire_gather(nxt, bn)

        return carry

    lax.fori_loop(0, ROWS_W // NBUF, step, 0)

    for u in range(NBUF):
        r = ROWS_W - NBUF + u
        wait_scatter(r, r % NBUF)


def kernel(input, lookup_table):
    return _embed(input.astype(jnp.int32), lookup_table)
